# Initial kernel scaffold; baseline (speedup 1.0000x reference)
import jax
import jax.numpy as jnp
from jax.experimental import pallas as pl


def kernel(up_w, up_b, c1_w, c1_b, c2_w, c2_b, bn1_g, bn1_b, bn2_g, bn2_b, x, skip):
    raise NotImplementedError("write your pallas kernel here")


# speedup vs baseline: 1.0836x; 1.0836x over previous
import functools

import jax
import jax.numpy as jnp
from jax import lax
from jax.experimental import pallas as pl
from jax.experimental.pallas import tpu as pltpu

_H = 64
_M = _H * _H
_CP = 128
_EPS = 1e-5


def _conv1_body(xu_ref, sk_ref, w_ref, bmap_ref, y_ref, st_ref, xpx, xps, col):
    cx = xpx.shape[-1]
    cs = xps.shape[-1]
    xpx[...] = jnp.zeros_like(xpx)
    xps[...] = jnp.zeros_like(xps)
    xpx[1:_H + 1, 1:_H + 1, :] = xu_ref[...]
    xps[1:_H + 1, 1:_H + 1, :] = sk_ref[...]
    kx = 9 * cx
    for t in range(9):
        dy, dx = divmod(t, 3)
        col[:, :, t * cx:(t + 1) * cx] = xpx[dy:dy + _H, dx:dx + _H, :]
        col[:, :, kx + t * cs:kx + (t + 1) * cs] = xps[dy:dy + _H, dx:dx + _H, :]
    p = col[...].reshape(_M, 9 * (cx + cs))
    acc = jnp.dot(p, w_ref[...], preferred_element_type=jnp.float32)
    y = jnp.maximum(acc + bmap_ref[...], 0.0)
    y_ref[...] = y.astype(jnp.bfloat16)
    st_ref[...] = jnp.concatenate(
        [jnp.sum(y, axis=0, keepdims=True),
         jnp.sum(y * y, axis=0, keepdims=True)], axis=0)


def _conv2_body(y1_ref, w_ref, bmap_ref, y_ref, st_ref, xpy, col):
    c = xpy.shape[-1]
    xpy[...] = jnp.zeros_like(xpy)
    xpy[1:_H + 1, 1:_H + 1, :] = y1_ref[...]
    for t in range(9):
        dy, dx = divmod(t, 3)
        col[:, :, t * c:(t + 1) * c] = xpy[dy:dy + _H, dx:dx + _H, :]
    p = col[...].reshape(_M, 9 * c)
    acc = jnp.dot(p, w_ref[...], preferred_element_type=jnp.float32)
    y = jnp.maximum(acc + bmap_ref[...], 0.0)
    y_ref[...] = y.astype(jnp.bfloat16)
    st_ref[...] = jnp.concatenate(
        [jnp.sum(y, axis=0, keepdims=True),
         jnp.sum(y * y, axis=0, keepdims=True)], axis=0)


def _conv1_call(xu, sk, w, bmap):
    n = xu.shape[0]
    cx, cs = xu.shape[-1], sk.shape[-1]
    k = 9 * (cx + cs)
    return pl.pallas_call(
        _conv1_body,
        out_shape=(jax.ShapeDtypeStruct((n, _M, _CP), jnp.bfloat16),
                   jax.ShapeDtypeStruct((n, 2, _CP), jnp.float32)),
        grid=(n,),
        in_specs=[
            pl.BlockSpec((None, _H, _H, cx), lambda i: (i, 0, 0, 0)),
            pl.BlockSpec((None, _H, _H, cs), lambda i: (i, 0, 0, 0)),
            pl.BlockSpec((k, _CP), lambda i: (0, 0)),
            pl.BlockSpec((_M, _CP), lambda i: (0, 0)),
        ],
        out_specs=(
            pl.BlockSpec((None, _M, _CP), lambda i: (i, 0, 0)),
            pl.BlockSpec((None, 2, _CP), lambda i: (i, 0, 0)),
        ),
        scratch_shapes=[
            pltpu.VMEM((_H + 2, _H + 2, cx), jnp.bfloat16),
            pltpu.VMEM((_H + 2, _H + 2, cs), jnp.bfloat16),
            pltpu.VMEM((_H, _H, k), jnp.bfloat16),
        ],
        compiler_params=pltpu.CompilerParams(
            dimension_semantics=("parallel",)),
    )(xu, sk, w, bmap)


def _conv2_call(y1, w, bmap):
    n = y1.shape[0]
    k = 9 * _CP
    return pl.pallas_call(
        _conv2_body,
        out_shape=(jax.ShapeDtypeStruct((n, _M, _CP), jnp.bfloat16),
                   jax.ShapeDtypeStruct((n, 2, _CP), jnp.float32)),
        grid=(n,),
        in_specs=[
            pl.BlockSpec((None, _H, _H, _CP), lambda i: (i, 0, 0, 0)),
            pl.BlockSpec((k, _CP), lambda i: (0, 0)),
            pl.BlockSpec((_M, _CP), lambda i: (0, 0)),
        ],
        out_specs=(
            pl.BlockSpec((None, _M, _CP), lambda i: (i, 0, 0)),
            pl.BlockSpec((None, 2, _CP), lambda i: (i, 0, 0)),
        ),
        scratch_shapes=[
            pltpu.VMEM((_H + 2, _H + 2, _CP), jnp.bfloat16),
            pltpu.VMEM((_H, _H, k), jnp.bfloat16),
        ],
        compiler_params=pltpu.CompilerParams(
            dimension_semantics=("parallel",)),
    )(y1, w, bmap)


def _up2x(v, axis):
    n = v.shape[axis]
    first = lax.slice_in_dim(v, 0, 1, axis=axis)
    last = lax.slice_in_dim(v, n - 1, n, axis=axis)
    prev = jnp.concatenate([first, lax.slice_in_dim(v, 0, n - 1, axis=axis)],
                           axis=axis)
    nxt = jnp.concatenate([lax.slice_in_dim(v, 1, n, axis=axis), last],
                          axis=axis)
    out = jnp.stack([0.75 * v + 0.25 * prev, 0.75 * v + 0.25 * nxt],
                    axis=axis + 1)
    shape = list(v.shape)
    shape[axis] = 2 * n
    return out.reshape(shape)


def _edge_bias_map(w_one, base):
    mask = jnp.pad(jnp.ones((_H, _H), jnp.float32), 1)
    acc = jnp.broadcast_to(base[None, None, :], (_H, _H, base.shape[0]))
    for t in range(9):
        dy, dx = divmod(t, 3)
        acc = acc + mask[dy:dy + _H, dx:dx + _H, None] * w_one[t][None, None, :]
    return acc.reshape(_M, base.shape[0])


def _pad_lanes(a, cp):
    return jnp.pad(a, [(0, 0)] * (a.ndim - 1) + [(0, cp - a.shape[-1])])


def _bn(stats, gamma, beta, count):
    mean = stats[0] / count
    var = jnp.maximum(stats[1] / count - mean * mean, 0.0)
    scale = gamma * lax.rsqrt(var + _EPS)
    shift = beta - mean * scale
    return scale, shift


@functools.partial(jax.jit, static_argnums=())
def kernel(up_w, up_b, c1_w, c1_b, c2_w, c2_b,
           bn1_g, bn1_b, bn2_g, bn2_b, x, skip):
    out_ch = up_w.shape[1]
    n = x.shape[0]
    count = n * _M

    x_nhwc = jnp.transpose(x, (0, 2, 3, 1)).astype(jnp.float32)
    sk = jnp.transpose(skip, (0, 2, 3, 1)).astype(jnp.bfloat16)
    xu = _up2x(_up2x(x_nhwc, 1), 2).astype(jnp.bfloat16)

    w_up_part = c1_w[:, :out_ch, :]
    w_sk_part = c1_w[:, out_ch:, :]
    w_x = jnp.einsum('ic,tco->tio', up_w, w_up_part)
    in_ch = up_w.shape[0]
    prev_ch = w_sk_part.shape[1]
    w1 = jnp.concatenate([w_x.reshape(9 * in_ch, out_ch),
                          w_sk_part.reshape(9 * prev_ch, out_ch)], axis=0)
    w1 = _pad_lanes(w1, _CP).astype(jnp.bfloat16)
    w_one1 = jnp.einsum('c,tco->to', up_b, w_up_part)
    bmap1 = _pad_lanes(_edge_bias_map(w_one1, c1_b), _CP)

    y1, st1 = _conv1_call(xu, sk, w1, bmap1)
    st1 = st1.sum(axis=0)[:, :out_ch]
    s1, t1 = _bn(st1, bn1_g, bn1_b, count)

    w_y = c2_w * s1[None, :, None]
    w2 = jnp.zeros((9, _CP, _CP), jnp.float32)
    w2 = w2.at[:, :out_ch, :out_ch].set(w_y).reshape(9 * _CP, _CP)
    w2 = w2.astype(jnp.bfloat16)
    w_one2 = jnp.einsum('c,tco->to', t1, c2_w)
    bmap2 = _pad_lanes(_edge_bias_map(w_one2, c2_b), _CP)

    y1_4d = y1.reshape(n, _H, _H, _CP)
    y2, st2 = _conv2_call(y1_4d, w2, bmap2)
    st2 = st2.sum(axis=0)[:, :out_ch]
    s2, t2 = _bn(st2, bn2_g, bn2_b, count)

    out = (y2.reshape(n, _H, _H, _CP)[..., :out_ch].astype(jnp.float32)
           * s2 + t2)
    return jnp.transpose(out, (0, 3, 1, 2))


# speedup vs baseline: 1.1004x; 1.0156x over previous
import functools

import jax
import jax.numpy as jnp
from jax import lax
from jax.experimental import pallas as pl
from jax.experimental.pallas import tpu as pltpu

_H = 64
_M = _H * _H
_CP = 128
_OC = 64
_EPS = 1e-5


def _edge_map(w_one_rows, base_row):
    top = w_one_rows[0] + w_one_rows[1] + w_one_rows[2]
    bot = w_one_rows[6] + w_one_rows[7] + w_one_rows[8]
    lef = w_one_rows[0] + w_one_rows[3] + w_one_rows[6]
    rig = w_one_rows[2] + w_one_rows[5] + w_one_rows[8]
    full = base_row + sum(w_one_rows[1:], w_one_rows[0])
    hh = lax.broadcasted_iota(jnp.int32, (_M, 1), 0) // _H
    ww = lax.broadcasted_iota(jnp.int32, (_M, 1), 0) % _H
    h0 = (hh == 0).astype(jnp.float32)
    h1 = (hh == _H - 1).astype(jnp.float32)
    w0 = (ww == 0).astype(jnp.float32)
    w1 = (ww == _H - 1).astype(jnp.float32)
    e = jnp.concatenate(
        [h0, h1, w0, w1, h0 * w0, h0 * w1, h1 * w0, h1 * w1], axis=1)
    corr = jnp.concatenate(
        [-top, -bot, -lef, -rig,
         w_one_rows[0], w_one_rows[2], w_one_rows[6], w_one_rows[8]], axis=0)
    return (jnp.broadcast_to(full, (_M, _OC))
            + jnp.dot(e, corr, preferred_element_type=jnp.float32))


def _prep1_body(up_w_ref, up_b_ref, c1_w_ref, c1_b_ref, w_ref, bmap_ref):
    w_ref[...] = jnp.zeros_like(w_ref)
    w_ones = []
    for t in range(9):
        wu = c1_w_ref[t, :_OC, :]
        w_ref[t * _CP:t * _CP + _CP, :_OC] = jnp.dot(
            up_w_ref[...], wu, preferred_element_type=jnp.float32
        ).astype(jnp.bfloat16)
        w_ref[9 * _CP + t * _OC:9 * _CP + (t + 1) * _OC, :_OC] = (
            c1_w_ref[t, _OC:, :].astype(jnp.bfloat16))
        w_ones.append(jnp.dot(up_b_ref[...], wu,
                              preferred_element_type=jnp.float32))
    bmap_ref[...] = jnp.zeros_like(bmap_ref)
    bmap_ref[:, :_OC] = _edge_map(w_ones, c1_b_ref[...])


def _prep1_call(up_w, up_b, c1_w, c1_b):
    k1 = 9 * (_CP + _OC)
    return pl.pallas_call(
        _prep1_body,
        out_shape=(jax.ShapeDtypeStruct((k1, _CP), jnp.bfloat16),
                   jax.ShapeDtypeStruct((_M, _CP), jnp.float32)),
        in_specs=[pl.BlockSpec(a.shape, lambda nd=a.ndim: (0,) * nd)
                  for a in (up_w, up_b, c1_w, c1_b)],
        out_specs=(pl.BlockSpec((k1, _CP), lambda: (0, 0)),
                   pl.BlockSpec((_M, _CP), lambda: (0, 0))),
    )(up_w, up_b, c1_w, c1_b)


def _prep2_body(st_ref, g1_ref, b1_ref, c2_w_ref, c2_b_ref, w_ref, bmap_ref,
                count):
    st = jnp.sum(st_ref[...], axis=0)
    mean = st[0:1, :_OC] / count
    var = jnp.maximum(st[1:2, :_OC] / count - mean * mean, 0.0)
    s1 = g1_ref[...] * lax.rsqrt(var + _EPS)
    t1 = b1_ref[...] - mean * s1
    s1c = jnp.transpose(s1)
    w_ref[...] = jnp.zeros_like(w_ref)
    w_ones = []
    for t in range(9):
        wt = c2_w_ref[t]
        w_ref[t * _CP:t * _CP + _OC, :_OC] = (wt * s1c).astype(jnp.bfloat16)
        w_ones.append(jnp.dot(t1, wt, preferred_element_type=jnp.float32))
    bmap_ref[...] = jnp.zeros_like(bmap_ref)
    bmap_ref[:, :_OC] = _edge_map(w_ones, c2_b_ref[...])


def _prep2_call(st_all, g1, b1, c2_w, c2_b, count):
    k2 = 9 * _CP
    return pl.pallas_call(
        functools.partial(_prep2_body, count=count),
        out_shape=(jax.ShapeDtypeStruct((k2, _CP), jnp.bfloat16),
                   jax.ShapeDtypeStruct((_M, _CP), jnp.float32)),
        in_specs=[pl.BlockSpec(a.shape, lambda nd=a.ndim: (0,) * nd)
                  for a in (st_all, g1, b1, c2_w, c2_b)],
        out_specs=(pl.BlockSpec((k2, _CP), lambda: (0, 0)),
                   pl.BlockSpec((_M, _CP), lambda: (0, 0))),
    )(st_all, g1, b1, c2_w, c2_b)


def _conv1_body(xu_ref, sk_ref, w_ref, bmap_ref, y_ref, st_ref, xpx, xps, col):
    cx = xpx.shape[-1]
    cs = xps.shape[-1]
    xpx[...] = jnp.zeros_like(xpx)
    xps[...] = jnp.zeros_like(xps)
    xpx[1:_H + 1, 1:_H + 1, :] = xu_ref[...]
    xps[1:_H + 1, 1:_H + 1, :] = sk_ref[...]
    kx = 9 * cx
    for t in range(9):
        dy, dx = divmod(t, 3)
        col[:, :, t * cx:(t + 1) * cx] = xpx[dy:dy + _H, dx:dx + _H, :]
        col[:, :, kx + t * cs:kx + (t + 1) * cs] = xps[dy:dy + _H, dx:dx + _H, :]
    p = col[...].reshape(_M, 9 * (cx + cs))
    acc = jnp.dot(p, w_ref[...], preferred_element_type=jnp.float32)
    y = jnp.maximum(acc + bmap_ref[...], 0.0)
    y_ref[...] = y.astype(jnp.bfloat16)
    st_ref[...] = jnp.concatenate(
        [jnp.sum(y, axis=0, keepdims=True),
         jnp.sum(y * y, axis=0, keepdims=True)], axis=0)


def _conv2_body(y1_ref, w_ref, bmap_ref, y_ref, st_ref, xpy, col):
    c = xpy.shape[-1]
    xpy[...] = jnp.zeros_like(xpy)
    xpy[1:_H + 1, 1:_H + 1, :] = y1_ref[...]
    for t in range(9):
        dy, dx = divmod(t, 3)
        col[:, :, t * c:(t + 1) * c] = xpy[dy:dy + _H, dx:dx + _H, :]
    p = col[...].reshape(_M, 9 * c)
    acc = jnp.dot(p, w_ref[...], preferred_element_type=jnp.float32)
    y = jnp.maximum(acc + bmap_ref[...], 0.0)
    y_ref[...] = y.astype(jnp.bfloat16)
    st_ref[...] = jnp.concatenate(
        [jnp.sum(y, axis=0, keepdims=True),
         jnp.sum(y * y, axis=0, keepdims=True)], axis=0)


def _conv1_call(xu, sk, w, bmap):
    n = xu.shape[0]
    cx, cs = xu.shape[-1], sk.shape[-1]
    k = 9 * (cx + cs)
    return pl.pallas_call(
        _conv1_body,
        out_shape=(jax.ShapeDtypeStruct((n, _M, _CP), jnp.bfloat16),
                   jax.ShapeDtypeStruct((n, 2, _CP), jnp.float32)),
        grid=(n,),
        in_specs=[
            pl.BlockSpec((None, _H, _H, cx), lambda i: (i, 0, 0, 0)),
            pl.BlockSpec((None, _H, _H, cs), lambda i: (i, 0, 0, 0)),
            pl.BlockSpec((k, _CP), lambda i: (0, 0)),
            pl.BlockSpec((_M, _CP), lambda i: (0, 0)),
        ],
        out_specs=(
            pl.BlockSpec((None, _M, _CP), lambda i: (i, 0, 0)),
            pl.BlockSpec((None, 2, _CP), lambda i: (i, 0, 0)),
        ),
        scratch_shapes=[
            pltpu.VMEM((_H + 2, _H + 2, cx), jnp.bfloat16),
            pltpu.VMEM((_H + 2, _H + 2, cs), jnp.bfloat16),
            pltpu.VMEM((_H, _H, k), jnp.bfloat16),
        ],
        compiler_params=pltpu.CompilerParams(
            dimension_semantics=("parallel",)),
    )(xu, sk, w, bmap)


def _conv2_call(y1, w, bmap):
    n = y1.shape[0]
    k = 9 * _CP
    return pl.pallas_call(
        _conv2_body,
        out_shape=(jax.ShapeDtypeStruct((n, _M, _CP), jnp.bfloat16),
                   jax.ShapeDtypeStruct((n, 2, _CP), jnp.float32)),
        grid=(n,),
        in_specs=[
            pl.BlockSpec((None, _H, _H, _CP), lambda i: (i, 0, 0, 0)),
            pl.BlockSpec((k, _CP), lambda i: (0, 0)),
            pl.BlockSpec((_M, _CP), lambda i: (0, 0)),
        ],
        out_specs=(
            pl.BlockSpec((None, _M, _CP), lambda i: (i, 0, 0)),
            pl.BlockSpec((None, 2, _CP), lambda i: (i, 0, 0)),
        ),
        scratch_shapes=[
            pltpu.VMEM((_H + 2, _H + 2, _CP), jnp.bfloat16),
            pltpu.VMEM((_H, _H, k), jnp.bfloat16),
        ],
        compiler_params=pltpu.CompilerParams(
            dimension_semantics=("parallel",)),
    )(y1, w, bmap)


def _up2x(v, axis):
    n = v.shape[axis]
    first = lax.slice_in_dim(v, 0, 1, axis=axis)
    last = lax.slice_in_dim(v, n - 1, n, axis=axis)
    prev = jnp.concatenate([first, lax.slice_in_dim(v, 0, n - 1, axis=axis)],
                           axis=axis)
    nxt = jnp.concatenate([lax.slice_in_dim(v, 1, n, axis=axis), last],
                          axis=axis)
    out = jnp.stack([0.75 * v + 0.25 * prev, 0.75 * v + 0.25 * nxt],
                    axis=axis + 1)
    shape = list(v.shape)
    shape[axis] = 2 * n
    return out.reshape(shape)


def kernel(up_w, up_b, c1_w, c1_b, c2_w, c2_b,
           bn1_g, bn1_b, bn2_g, bn2_b, x, skip):
    n = x.shape[0]
    count = float(n * _M)

    x_nhwc = jnp.transpose(x, (0, 2, 3, 1)).astype(jnp.float32)
    sk = jnp.transpose(skip, (0, 2, 3, 1)).astype(jnp.bfloat16)
    xu = _up2x(_up2x(x_nhwc, 1), 2).astype(jnp.bfloat16)

    w1, bmap1 = _prep1_call(up_w, up_b[None, :], c1_w, c1_b[None, :])
    y1, st1 = _conv1_call(xu, sk, w1, bmap1)

    w2, bmap2 = _prep2_call(st1, bn1_g[None, :], bn1_b[None, :],
                            c2_w, c2_b[None, :], count)
    y2, st2 = _conv2_call(y1.reshape(n, _H, _H, _CP), w2, bmap2)

    st2s = st2.sum(axis=0)[:, :_OC]
    mean = st2s[0] / count
    var = jnp.maximum(st2s[1] / count - mean * mean, 0.0)
    s2 = bn2_g * lax.rsqrt(var + _EPS)
    t2 = bn2_b - mean * s2
    out = (y2.reshape(n, _H, _H, _CP)[..., :_OC].astype(jnp.float32)
           * s2 + t2)
    return jnp.transpose(out, (0, 3, 1, 2))


# speedup vs baseline: 1.1542x; 1.0488x over previous
import functools

import jax
import jax.numpy as jnp
from jax import lax
from jax.experimental import pallas as pl
from jax.experimental.pallas import tpu as pltpu

_H = 64
_M = _H * _H
_OC = 64
_IC = 128
_K1 = 9 * (_IC + _OC)
_K2 = 9 * _OC
_EPS = 1e-5


def _edge_map(w_one_rows, base_row):
    top = w_one_rows[0] + w_one_rows[1] + w_one_rows[2]
    bot = w_one_rows[6] + w_one_rows[7] + w_one_rows[8]
    lef = w_one_rows[0] + w_one_rows[3] + w_one_rows[6]
    rig = w_one_rows[2] + w_one_rows[5] + w_one_rows[8]
    full = base_row + sum(w_one_rows[1:], w_one_rows[0])
    hh = lax.broadcasted_iota(jnp.int32, (_M, 1), 0) // _H
    ww = lax.broadcasted_iota(jnp.int32, (_M, 1), 0) % _H
    h0 = (hh == 0).astype(jnp.float32)
    h1 = (hh == _H - 1).astype(jnp.float32)
    w0 = (ww == 0).astype(jnp.float32)
    w1 = (ww == _H - 1).astype(jnp.float32)
    e = jnp.concatenate(
        [h0, h1, w0, w1, h0 * w0, h0 * w1, h1 * w0, h1 * w1], axis=1)
    corr = jnp.concatenate(
        [-top, -bot, -lef, -rig,
         w_one_rows[0], w_one_rows[2], w_one_rows[6], w_one_rows[8]], axis=0)
    return (jnp.broadcast_to(full, (_M, _OC))
            + jnp.dot(e, corr, preferred_element_type=jnp.float32))


def _bn_rows(st, g_row, b_row, count):
    mean = st[0:1] / count
    var = jnp.maximum(st[1:2] / count - mean * mean, 0.0)
    scale = g_row * lax.rsqrt(var + _EPS)
    shift = b_row - mean * scale
    return scale, shift


def _prep1_body(up_w_ref, up_b_ref, c1_w_ref, c1_b_ref, w_ref, bmap_ref):
    w_ones = []
    for t in range(9):
        wu = c1_w_ref[t, :_OC, :]
        w_ref[t * _IC:(t + 1) * _IC, :] = jnp.dot(
            up_w_ref[...], wu, preferred_element_type=jnp.float32
        ).astype(jnp.bfloat16)
        w_ref[9 * _IC + t * _OC:9 * _IC + (t + 1) * _OC, :] = (
            c1_w_ref[t, _OC:, :].astype(jnp.bfloat16))
        w_ones.append(jnp.dot(up_b_ref[...], wu,
                              preferred_element_type=jnp.float32))
    bmap_ref[...] = _edge_map(w_ones, c1_b_ref[...])


def _prep1_call(up_w, up_b, c1_w, c1_b):
    return pl.pallas_call(
        _prep1_body,
        out_shape=(jax.ShapeDtypeStruct((_K1, _OC), jnp.bfloat16),
                   jax.ShapeDtypeStruct((_M, _OC), jnp.float32)),
        in_specs=[pl.BlockSpec(a.shape, lambda nd=a.ndim: (0,) * nd)
                  for a in (up_w, up_b, c1_w, c1_b)],
        out_specs=(pl.BlockSpec((_K1, _OC), lambda: (0, 0)),
                   pl.BlockSpec((_M, _OC), lambda: (0, 0))),
    )(up_w, up_b, c1_w, c1_b)


def _prep2_body(st_ref, g1_ref, b1_ref, c2_w_ref, c2_b_ref, w_ref, bmap_ref,
                count):
    st = jnp.sum(st_ref[...], axis=0)
    s1, t1 = _bn_rows(st, g1_ref[...], b1_ref[...], count)
    s1c = jnp.transpose(s1)
    w_ones = []
    for t in range(9):
        wt = c2_w_ref[t]
        w_ref[t * _OC:(t + 1) * _OC, :] = (wt * s1c).astype(jnp.bfloat16)
        w_ones.append(jnp.dot(t1, wt, preferred_element_type=jnp.float32))
    bmap_ref[...] = _edge_map(w_ones, c2_b_ref[...])


def _prep2_call(st_all, g1, b1, c2_w, c2_b, count):
    return pl.pallas_call(
        functools.partial(_prep2_body, count=count),
        out_shape=(jax.ShapeDtypeStruct((_K2, _OC), jnp.bfloat16),
                   jax.ShapeDtypeStruct((_M, _OC), jnp.float32)),
        in_specs=[pl.BlockSpec(a.shape, lambda nd=a.ndim: (0,) * nd)
                  for a in (st_all, g1, b1, c2_w, c2_b)],
        out_specs=(pl.BlockSpec((_K2, _OC), lambda: (0, 0)),
                   pl.BlockSpec((_M, _OC), lambda: (0, 0))),
    )(st_all, g1, b1, c2_w, c2_b)


def _upsample2x(xt):
    h = xt.shape[0]
    prev = jnp.concatenate([xt[0:1], xt[:-1]], axis=0)
    nxt = jnp.concatenate([xt[1:], xt[-1:]], axis=0)
    xh = jnp.stack([0.75 * xt + 0.25 * prev, 0.75 * xt + 0.25 * nxt],
                   axis=1).reshape(2 * h, h, xt.shape[2])
    prevw = jnp.concatenate([xh[:, 0:1], xh[:, :-1]], axis=1)
    nxtw = jnp.concatenate([xh[:, 1:], xh[:, -1:]], axis=1)
    return jnp.stack([0.75 * xh + 0.25 * prevw, 0.75 * xh + 0.25 * nxtw],
                     axis=2).reshape(2 * h, 2 * h, xt.shape[2])


def _relu_stats_store(acc, bmap_ref, y_ref, st_ref):
    y = jnp.maximum(acc + bmap_ref[...], 0.0)
    y_ref[...] = y.astype(jnp.bfloat16)
    st_ref[...] = jnp.concatenate(
        [jnp.sum(y, axis=0, keepdims=True),
         jnp.sum(y * y, axis=0, keepdims=True)], axis=0)


def _conv1_body(x_ref, sk_ref, w_ref, bmap_ref, y_ref, st_ref, xpx, xps, col):
    xt = jnp.transpose(x_ref[...], (1, 2, 0))
    xu = _upsample2x(xt)
    xpx[...] = jnp.zeros_like(xpx)
    xps[...] = jnp.zeros_like(xps)
    xpx[1:_H + 1, 1:_H + 1, :] = xu.astype(jnp.bfloat16)
    xps[1:_H + 1, 1:_H + 1, :] = jnp.transpose(
        sk_ref[...], (1, 2, 0)).astype(jnp.bfloat16)
    for t in range(9):
        dy, dx = divmod(t, 3)
        col[:, :, t * _IC:(t + 1) * _IC] = xpx[dy:dy + _H, dx:dx + _H, :]
        col[:, :, 9 * _IC + t * _OC:9 * _IC + (t + 1) * _OC] = (
            xps[dy:dy + _H, dx:dx + _H, :])
    acc = jnp.dot(col[...].reshape(_M, _K1), w_ref[...],
                  preferred_element_type=jnp.float32)
    _relu_stats_store(acc, bmap_ref, y_ref, st_ref)


def _conv2_body(y1_ref, w_ref, bmap_ref, y_ref, st_ref, xpy, col):
    xpy[...] = jnp.zeros_like(xpy)
    xpy[1:_H + 1, 1:_H + 1, :] = y1_ref[...]
    for t in range(9):
        dy, dx = divmod(t, 3)
        col[:, :, t * _OC:(t + 1) * _OC] = xpy[dy:dy + _H, dx:dx + _H, :]
    acc = jnp.dot(col[...].reshape(_M, _K2), w_ref[...],
                  preferred_element_type=jnp.float32)
    _relu_stats_store(acc, bmap_ref, y_ref, st_ref)


def _conv1_call(x, skip, w, bmap):
    n = x.shape[0]
    return pl.pallas_call(
        _conv1_body,
        out_shape=(jax.ShapeDtypeStruct((n, _M, _OC), jnp.bfloat16),
                   jax.ShapeDtypeStruct((n, 2, _OC), jnp.float32)),
        grid=(n,),
        in_specs=[
            pl.BlockSpec((None, _IC, _H // 2, _H // 2), lambda i: (i, 0, 0, 0)),
            pl.BlockSpec((None, _OC, _H, _H), lambda i: (i, 0, 0, 0)),
            pl.BlockSpec((_K1, _OC), lambda i: (0, 0)),
            pl.BlockSpec((_M, _OC), lambda i: (0, 0)),
        ],
        out_specs=(
            pl.BlockSpec((None, _M, _OC), lambda i: (i, 0, 0)),
            pl.BlockSpec((None, 2, _OC), lambda i: (i, 0, 0)),
        ),
        scratch_shapes=[
            pltpu.VMEM((_H + 2, _H + 2, _IC), jnp.bfloat16),
            pltpu.VMEM((_H + 2, _H + 2, _OC), jnp.bfloat16),
            pltpu.VMEM((_H, _H, _K1), jnp.bfloat16),
        ],
        compiler_params=pltpu.CompilerParams(
            dimension_semantics=("parallel",)),
    )(x, skip, w, bmap)


def _conv2_call(y1, w, bmap):
    n = y1.shape[0]
    return pl.pallas_call(
        _conv2_body,
        out_shape=(jax.ShapeDtypeStruct((n, _M, _OC), jnp.bfloat16),
                   jax.ShapeDtypeStruct((n, 2, _OC), jnp.float32)),
        grid=(n,),
        in_specs=[
            pl.BlockSpec((None, _H, _H, _OC), lambda i: (i, 0, 0, 0)),
            pl.BlockSpec((_K2, _OC), lambda i: (0, 0)),
            pl.BlockSpec((_M, _OC), lambda i: (0, 0)),
        ],
        out_specs=(
            pl.BlockSpec((None, _M, _OC), lambda i: (i, 0, 0)),
            pl.BlockSpec((None, 2, _OC), lambda i: (i, 0, 0)),
        ),
        scratch_shapes=[
            pltpu.VMEM((_H + 2, _H + 2, _OC), jnp.bfloat16),
            pltpu.VMEM((_H, _H, _K2), jnp.bfloat16),
        ],
        compiler_params=pltpu.CompilerParams(
            dimension_semantics=("parallel",)),
    )(y1, w, bmap)


def _epi_body(st_ref, g2_ref, b2_ref, y2_ref, o_ref, count):
    st = jnp.sum(st_ref[...], axis=0)
    s2, t2 = _bn_rows(st, g2_ref[...], b2_ref[...], count)
    z = y2_ref[...].astype(jnp.float32) * s2 + t2
    o_ref[...] = jnp.transpose(z)


def _epi_call(st_all, g2, b2, y2, count):
    n = y2.shape[0]
    return pl.pallas_call(
        functools.partial(_epi_body, count=count),
        out_shape=jax.ShapeDtypeStruct((n, _OC, _M), jnp.float32),
        grid=(n,),
        in_specs=[
            pl.BlockSpec((n, 2, _OC), lambda i: (0, 0, 0)),
            pl.BlockSpec((1, _OC), lambda i: (0, 0)),
            pl.BlockSpec((1, _OC), lambda i: (0, 0)),
            pl.BlockSpec((None, _M, _OC), lambda i: (i, 0, 0)),
        ],
        out_specs=pl.BlockSpec((None, _OC, _M), lambda i: (i, 0, 0)),
        compiler_params=pltpu.CompilerParams(
            dimension_semantics=("parallel",)),
    )(st_all, g2, b2, y2)


def kernel(up_w, up_b, c1_w, c1_b, c2_w, c2_b,
           bn1_g, bn1_b, bn2_g, bn2_b, x, skip):
    n = x.shape[0]
    count = float(n * _M)

    w1, bmap1 = _prep1_call(up_w, up_b[None, :], c1_w, c1_b[None, :])
    y1, st1 = _conv1_call(x, skip, w1, bmap1)

    w2, bmap2 = _prep2_call(st1, bn1_g[None, :], bn1_b[None, :],
                            c2_w, c2_b[None, :], count)
    y2, st2 = _conv2_call(y1.reshape(n, _H, _H, _OC), w2, bmap2)

    out = _epi_call(st2, bn2_g[None, :], bn2_b[None, :], y2, count)
    return out.reshape(n, _OC, _H, _H)


# speedup vs baseline: 1.2079x; 1.0465x over previous
import functools

import jax
import jax.numpy as jnp
from jax import lax
from jax.experimental import pallas as pl
from jax.experimental.pallas import tpu as pltpu

_H = 64
_M = _H * _H
_OC = 64
_IC = 128
_K1 = 9 * (_IC + _OC)
_K2 = 9 * _OC
_EPS = 1e-5


def _edge_map(w_one_rows, base_row):
    top = w_one_rows[0] + w_one_rows[1] + w_one_rows[2]
    bot = w_one_rows[6] + w_one_rows[7] + w_one_rows[8]
    lef = w_one_rows[0] + w_one_rows[3] + w_one_rows[6]
    rig = w_one_rows[2] + w_one_rows[5] + w_one_rows[8]
    full = base_row + sum(w_one_rows[1:], w_one_rows[0])
    hh = lax.broadcasted_iota(jnp.int32, (_M, 1), 0) // _H
    ww = lax.broadcasted_iota(jnp.int32, (_M, 1), 0) % _H
    h0 = (hh == 0).astype(jnp.float32)
    h1 = (hh == _H - 1).astype(jnp.float32)
    w0 = (ww == 0).astype(jnp.float32)
    w1 = (ww == _H - 1).astype(jnp.float32)
    e = jnp.concatenate(
        [h0, h1, w0, w1, h0 * w0, h0 * w1, h1 * w0, h1 * w1], axis=1)
    corr = jnp.concatenate(
        [-top, -bot, -lef, -rig,
         w_one_rows[0], w_one_rows[2], w_one_rows[6], w_one_rows[8]], axis=0)
    return (jnp.broadcast_to(full, (_M, _OC))
            + jnp.dot(e, corr, preferred_element_type=jnp.float32))


def _bn_rows(st, g_row, b_row, count):
    mean = st[0:1] / count
    var = jnp.maximum(st[1:2] / count - mean * mean, 0.0)
    scale = g_row * lax.rsqrt(var + _EPS)
    shift = b_row - mean * scale
    return scale, shift


def _prep1_body(up_w_ref, up_b_ref, c1_w_ref, c1_b_ref, w_ref, bmap_ref):
    w_ones = []
    for t in range(9):
        wu = c1_w_ref[t, :_OC, :]
        w_ref[t * _IC:(t + 1) * _IC, :] = jnp.dot(
            up_w_ref[...], wu, preferred_element_type=jnp.float32
        ).astype(jnp.bfloat16)
        w_ref[9 * _IC + t * _OC:9 * _IC + (t + 1) * _OC, :] = (
            c1_w_ref[t, _OC:, :].astype(jnp.bfloat16))
        w_ones.append(jnp.dot(up_b_ref[...], wu,
                              preferred_element_type=jnp.float32))
    bmap_ref[...] = _edge_map(w_ones, c1_b_ref[...])


def _prep1_call(up_w, up_b, c1_w, c1_b):
    return pl.pallas_call(
        _prep1_body,
        out_shape=(jax.ShapeDtypeStruct((_K1, _OC), jnp.bfloat16),
                   jax.ShapeDtypeStruct((_M, _OC), jnp.float32)),
        in_specs=[pl.BlockSpec(a.shape, lambda nd=a.ndim: (0,) * nd)
                  for a in (up_w, up_b, c1_w, c1_b)],
        out_specs=(pl.BlockSpec((_K1, _OC), lambda: (0, 0)),
                   pl.BlockSpec((_M, _OC), lambda: (0, 0))),
    )(up_w, up_b, c1_w, c1_b)


def _prep2_body(st_ref, g1_ref, b1_ref, c2_w_ref, c2_b_ref, w_ref, bmap_ref,
                count):
    st = jnp.sum(st_ref[...], axis=0)
    s1, t1 = _bn_rows(st, g1_ref[...], b1_ref[...], count)
    s1c = jnp.transpose(s1)
    w_ones = []
    for t in range(9):
        wt = c2_w_ref[t]
        w_ref[t * _OC:(t + 1) * _OC, :] = (wt * s1c).astype(jnp.bfloat16)
        w_ones.append(jnp.dot(t1, wt, preferred_element_type=jnp.float32))
    bmap_ref[...] = _edge_map(w_ones, c2_b_ref[...])


def _prep2_call(st_all, g1, b1, c2_w, c2_b, count):
    return pl.pallas_call(
        functools.partial(_prep2_body, count=count),
        out_shape=(jax.ShapeDtypeStruct((_K2, _OC), jnp.bfloat16),
                   jax.ShapeDtypeStruct((_M, _OC), jnp.float32)),
        in_specs=[pl.BlockSpec(a.shape, lambda nd=a.ndim: (0,) * nd)
                  for a in (st_all, g1, b1, c2_w, c2_b)],
        out_specs=(pl.BlockSpec((_K2, _OC), lambda: (0, 0)),
                   pl.BlockSpec((_M, _OC), lambda: (0, 0))),
    )(st_all, g1, b1, c2_w, c2_b)


def _upsample2x(xt):
    h = xt.shape[0]
    prev = jnp.concatenate([xt[0:1], xt[:-1]], axis=0)
    nxt = jnp.concatenate([xt[1:], xt[-1:]], axis=0)
    xh = jnp.stack([0.75 * xt + 0.25 * prev, 0.75 * xt + 0.25 * nxt],
                   axis=1).reshape(2 * h, h, xt.shape[2])
    prevw = jnp.concatenate([xh[:, 0:1], xh[:, :-1]], axis=1)
    nxtw = jnp.concatenate([xh[:, 1:], xh[:, -1:]], axis=1)
    return jnp.stack([0.75 * xh + 0.25 * prevw, 0.75 * xh + 0.25 * nxtw],
                     axis=2).reshape(2 * h, 2 * h, xt.shape[2])


def _relu_stats_store(acc, bmap_ref, y_ref, st_ref):
    y = jnp.maximum(acc + bmap_ref[...], 0.0)
    y_ref[...] = y.astype(jnp.bfloat16)
    st_ref[...] = jnp.concatenate(
        [jnp.sum(y, axis=0, keepdims=True),
         jnp.sum(y * y, axis=0, keepdims=True)], axis=0)


_B = 4
_B1 = 2


def _conv1_body(x_ref, sk_ref, w_ref, bmap_ref, y_ref, st_ref, xpx, xps, col):
    xpx[:, 0, :] = jnp.zeros_like(xpx[:, 0, :])
    xpx[:, _H + 1, :] = jnp.zeros_like(xpx[:, _H + 1, :])
    xpx[0, :, :] = jnp.zeros_like(xpx[0, :, :])
    xpx[_H + 1, :, :] = jnp.zeros_like(xpx[_H + 1, :, :])
    xps[:, 0, :] = jnp.zeros_like(xps[:, 0, :])
    xps[:, _H + 1, :] = jnp.zeros_like(xps[:, _H + 1, :])
    xps[0, :, :] = jnp.zeros_like(xps[0, :, :])
    xps[_H + 1, :, :] = jnp.zeros_like(xps[_H + 1, :, :])
    for img in range(_B1):
        xt = jnp.transpose(x_ref[img], (1, 2, 0))
        xu = _upsample2x(xt)
        xpx[1:_H + 1, 1:_H + 1, :] = xu.astype(jnp.bfloat16)
        xps[1:_H + 1, 1:_H + 1, :] = jnp.transpose(
            sk_ref[img], (1, 2, 0)).astype(jnp.bfloat16)
        for t in range(9):
            dy, dx = divmod(t, 3)
            col[:, :, t * _IC:(t + 1) * _IC] = xpx[dy:dy + _H, dx:dx + _H, :]
            col[:, :, 9 * _IC + t * _OC:9 * _IC + (t + 1) * _OC] = (
                xps[dy:dy + _H, dx:dx + _H, :])
        acc = jnp.dot(col[...].reshape(_M, _K1), w_ref[...],
                      preferred_element_type=jnp.float32)
        _relu_stats_store(acc, bmap_ref, y_ref.at[img], st_ref.at[img])


def _conv2_body(y1_ref, w_ref, bmap_ref, y_ref, st_ref, xpy, col):
    xpy[:, 0, :] = jnp.zeros_like(xpy[:, 0, :])
    xpy[:, _H + 1, :] = jnp.zeros_like(xpy[:, _H + 1, :])
    xpy[0, :, :] = jnp.zeros_like(xpy[0, :, :])
    xpy[_H + 1, :, :] = jnp.zeros_like(xpy[_H + 1, :, :])
    for img in range(_B):
        xpy[1:_H + 1, 1:_H + 1, :] = y1_ref[img]
        for t in range(9):
            dy, dx = divmod(t, 3)
            col[:, :, t * _OC:(t + 1) * _OC] = xpy[dy:dy + _H, dx:dx + _H, :]
        acc = jnp.dot(col[...].reshape(_M, _K2), w_ref[...],
                      preferred_element_type=jnp.float32)
        _relu_stats_store(acc, bmap_ref, y_ref.at[img], st_ref.at[img])


def _conv1_call(x, skip, w, bmap):
    n = x.shape[0]
    return pl.pallas_call(
        _conv1_body,
        out_shape=(jax.ShapeDtypeStruct((n, _M, _OC), jnp.bfloat16),
                   jax.ShapeDtypeStruct((n, 2, _OC), jnp.float32)),
        grid=(n // _B1,),
        in_specs=[
            pl.BlockSpec((_B1, _IC, _H // 2, _H // 2), lambda i: (i, 0, 0, 0)),
            pl.BlockSpec((_B1, _OC, _H, _H), lambda i: (i, 0, 0, 0)),
            pl.BlockSpec((_K1, _OC), lambda i: (0, 0)),
            pl.BlockSpec((_M, _OC), lambda i: (0, 0)),
        ],
        out_specs=(
            pl.BlockSpec((_B1, _M, _OC), lambda i: (i, 0, 0)),
            pl.BlockSpec((_B1, 2, _OC), lambda i: (i, 0, 0)),
        ),
        scratch_shapes=[
            pltpu.VMEM((_H + 2, _H + 2, _IC), jnp.bfloat16),
            pltpu.VMEM((_H + 2, _H + 2, _OC), jnp.bfloat16),
            pltpu.VMEM((_H, _H, _K1), jnp.bfloat16),
        ],
        compiler_params=pltpu.CompilerParams(
            dimension_semantics=("parallel",)),
    )(x, skip, w, bmap)


def _conv2_call(y1, w, bmap):
    n = y1.shape[0]
    return pl.pallas_call(
        _conv2_body,
        out_shape=(jax.ShapeDtypeStruct((n, _M, _OC), jnp.bfloat16),
                   jax.ShapeDtypeStruct((n, 2, _OC), jnp.float32)),
        grid=(n // _B,),
        in_specs=[
            pl.BlockSpec((_B, _H, _H, _OC), lambda i: (i, 0, 0, 0)),
            pl.BlockSpec((_K2, _OC), lambda i: (0, 0)),
            pl.BlockSpec((_M, _OC), lambda i: (0, 0)),
        ],
        out_specs=(
            pl.BlockSpec((_B, _M, _OC), lambda i: (i, 0, 0)),
            pl.BlockSpec((_B, 2, _OC), lambda i: (i, 0, 0)),
        ),
        scratch_shapes=[
            pltpu.VMEM((_H + 2, _H + 2, _OC), jnp.bfloat16),
            pltpu.VMEM((_H, _H, _K2), jnp.bfloat16),
        ],
        compiler_params=pltpu.CompilerParams(
            dimension_semantics=("parallel",)),
    )(y1, w, bmap)


def _epi_body(st_ref, g2_ref, b2_ref, y2_ref, o_ref, count):
    st = jnp.sum(st_ref[...], axis=0)
    s2, t2 = _bn_rows(st, g2_ref[...], b2_ref[...], count)
    for img in range(_B):
        z = y2_ref[img].astype(jnp.float32) * s2 + t2
        o_ref[img] = jnp.transpose(z.reshape(_H, _H, _OC), (2, 0, 1))


def _epi_call(st_all, g2, b2, y2, count):
    n = y2.shape[0]
    return pl.pallas_call(
        functools.partial(_epi_body, count=count),
        out_shape=jax.ShapeDtypeStruct((n, _OC, _H, _H), jnp.float32),
        grid=(n // _B,),
        in_specs=[
            pl.BlockSpec((n, 2, _OC), lambda i: (0, 0, 0)),
            pl.BlockSpec((1, _OC), lambda i: (0, 0)),
            pl.BlockSpec((1, _OC), lambda i: (0, 0)),
            pl.BlockSpec((_B, _M, _OC), lambda i: (i, 0, 0)),
        ],
        out_specs=pl.BlockSpec((_B, _OC, _H, _H), lambda i: (i, 0, 0, 0)),
        compiler_params=pltpu.CompilerParams(
            dimension_semantics=("parallel",)),
    )(st_all, g2, b2, y2)


def kernel(up_w, up_b, c1_w, c1_b, c2_w, c2_b,
           bn1_g, bn1_b, bn2_g, bn2_b, x, skip):
    n = x.shape[0]
    count = float(n * _M)

    w1, bmap1 = _prep1_call(up_w, up_b[None, :], c1_w, c1_b[None, :])
    y1, st1 = _conv1_call(x, skip, w1, bmap1)

    w2, bmap2 = _prep2_call(st1, bn1_g[None, :], bn1_b[None, :],
                            c2_w, c2_b[None, :], count)
    y2, st2 = _conv2_call(y1.reshape(n, _H, _H, _OC), w2, bmap2)

    return _epi_call(st2, bn2_g[None, :], bn2_b[None, :], y2, count)
